# Initial kernel scaffold; baseline (speedup 1.0000x reference)
import jax
import jax.numpy as jnp
from jax.experimental import pallas as pl


def kernel(x, table):
    raise NotImplementedError("write your pallas kernel here")


# speedup vs baseline: 1.1135x; 1.1135x over previous
import functools
import jax
import jax.numpy as jnp
from jax import lax
from jax.experimental import pallas as pl
from jax.experimental.pallas import tpu as pltpu
from jax.experimental.pallas import tpu_sc as plsc

_EMB_DIM = 32

_info = plsc.get_sparse_core_info()
_NC, _NS = _info.num_cores, _info.num_subcores
_NW = _NC * _NS

_NBUF = 4


def _build_gather(B: int, D: int, chunk: int):
  assert B % _NW == 0
  b_per_w = B // _NW
  assert b_per_w % chunk == 0 and chunk % 8 == 0
  n_chunks = b_per_w // chunk
  assert n_chunks % _NBUF == 0 and n_chunks // _NBUF >= 3
  n_groups = n_chunks // _NBUF
  mesh = plsc.VectorSubcoreMesh(core_axis_name="c", subcore_axis_name="s")

  @functools.partial(
      pl.kernel,
      mesh=mesh,
      out_type=jax.ShapeDtypeStruct((B, D), jnp.float32),
      scratch_types=[
          pltpu.VMEM((b_per_w,), jnp.int32),
          [pltpu.VMEM((chunk, D), jnp.float32) for _ in range(_NBUF)],
          [pltpu.SemaphoreType.DMA for _ in range(_NBUF)],
          [pltpu.SemaphoreType.DMA for _ in range(_NBUF)],
      ],
      compiler_params=pltpu.CompilerParams(use_tc_tiling_on_sc=False),
  )
  def gather(idx_hbm, table_hbm, out_hbm, idx_v, rows, gsem, osem):
    wid = lax.axis_index("s") * _NC + lax.axis_index("c")
    base = wid * b_per_w
    pltpu.sync_copy(idx_hbm.at[pl.ds(base, b_per_w)], idx_v)

    def start_gather(c, b):
      pltpu.async_copy(
          table_hbm.at[idx_v.at[pl.ds(c * chunk, chunk)]], rows[b], gsem[b])

    def start_out(c, b):
      pltpu.async_copy(
          rows[b], out_hbm.at[pl.ds(base + c * chunk, chunk)], osem[b])

    def wait_gather(b):
      pltpu.make_async_copy(table_hbm.at[idx_v.at[pl.ds(0, chunk)]],
                            rows[b], gsem[b]).wait()

    def wait_out(b):
      pltpu.make_async_copy(rows[b], out_hbm.at[pl.ds(0, chunk)],
                            osem[b]).wait()

    start_gather(0, 0)
    start_gather(1, 1)
    for b in range(_NBUF):
      wait_gather(b)
      start_out(b, b)
      if b < 2:
        start_gather(b + 2, (b + 2) % _NBUF)
      else:
        wait_out((b + 2) % _NBUF)
        start_gather(b + 2, (b + 2) % _NBUF)

    def group(j, carry):
      c0 = j * _NBUF
      for b in range(_NBUF):
        wait_gather(b)
        start_out(c0 + b, b)
        wait_out((b + 2) % _NBUF)
        start_gather(c0 + b + 2, (b + 2) % _NBUF)
      return carry

    lax.fori_loop(1, n_groups - 1, group, 0)

    c0 = (n_groups - 1) * _NBUF
    for b in range(_NBUF):
      wait_gather(b)
      start_out(c0 + b, b)
      if b < 2:
        wait_out((b + 2) % _NBUF)
        start_gather(c0 + b + 2, (b + 2) % _NBUF)
    for b in range(_NBUF):
      wait_out(b)

  return gather


_gather = _build_gather(16384 * 50, _EMB_DIM, chunk=640)


def kernel(x, table):
  idx = x.reshape(-1).astype(jnp.int32)
  out = _gather(idx, table)
  return out.reshape(x.shape + (_EMB_DIM,))


# speedup vs baseline: 1.1139x; 1.0003x over previous
import functools
import jax
import jax.numpy as jnp
from jax import lax
from jax.experimental import pallas as pl
from jax.experimental.pallas import tpu as pltpu
from jax.experimental.pallas import tpu_sc as plsc

_EMB_DIM = 32

_info = plsc.get_sparse_core_info()
_NC, _NS = _info.num_cores, _info.num_subcores
_NW = _NC * _NS


def _build_gather(B: int, D: int, chunk: int, nbuf: int, k: int):
  assert B % _NW == 0
  b_per_w = B // _NW
  assert b_per_w % chunk == 0 and chunk % 8 == 0
  n_chunks = b_per_w // chunk
  assert 1 <= k < nbuf
  assert n_chunks % nbuf == 0 and n_chunks // nbuf >= 3
  n_groups = n_chunks // nbuf
  mesh = plsc.VectorSubcoreMesh(core_axis_name="c", subcore_axis_name="s")

  @functools.partial(
      pl.kernel,
      mesh=mesh,
      out_type=jax.ShapeDtypeStruct((B, D), jnp.float32),
      scratch_types=[
          pltpu.VMEM((b_per_w,), jnp.int32),
          [pltpu.VMEM((chunk, D), jnp.float32) for _ in range(nbuf)],
          [pltpu.SemaphoreType.DMA for _ in range(nbuf)],
          [pltpu.SemaphoreType.DMA for _ in range(nbuf)],
      ],
      compiler_params=pltpu.CompilerParams(use_tc_tiling_on_sc=False),
  )
  def gather(idx_hbm, table_hbm, out_hbm, idx_v, rows, gsem, osem):
    wid = lax.axis_index("s") * _NC + lax.axis_index("c")
    base = wid * b_per_w
    pltpu.sync_copy(idx_hbm.at[pl.ds(base, b_per_w)], idx_v)

    def start_gather(c, b):
      pltpu.async_copy(
          table_hbm.at[idx_v.at[pl.ds(c * chunk, chunk)]], rows[b], gsem[b])

    def start_out(c, b):
      pltpu.async_copy(
          rows[b], out_hbm.at[pl.ds(base + c * chunk, chunk)], osem[b])

    def wait_gather(b):
      pltpu.make_async_copy(table_hbm.at[idx_v.at[pl.ds(0, chunk)]],
                            rows[b], gsem[b]).wait()

    def wait_out(b):
      pltpu.make_async_copy(rows[b], out_hbm.at[pl.ds(0, chunk)],
                            osem[b]).wait()

    def step(c, b, issue_gather, first_round):
      wait_gather(b)
      start_out(c, b)
      if issue_gather:
        b2 = (b + k) % nbuf
        if not first_round:
          wait_out(b2)
        start_gather(c + k, b2)

    for c in range(k):
      start_gather(c, c % nbuf)

    for b in range(nbuf):
      step(b, b, True, first_round=(b + k < nbuf))

    def group(j, carry):
      c0 = j * nbuf
      for b in range(nbuf):
        step(c0 + b, b, True, False)
      return carry

    lax.fori_loop(1, n_groups - 1, group, 0)

    c0 = (n_groups - 1) * nbuf
    for b in range(nbuf):
      step(c0 + b, b, b < nbuf - k, False)

    for b in range(nbuf):
      wait_out(b)

  return gather


_gather = _build_gather(16384 * 50, _EMB_DIM, chunk=320, nbuf=8, k=6)


def kernel(x, table):
  idx = x.reshape(-1).astype(jnp.int32)
  out = _gather(idx, table)
  return out.reshape(x.shape + (_EMB_DIM,))
